# baseline (speedup 1.0000x reference)
import functools

import jax
import jax.numpy as jnp
from jax import lax
from jax.experimental import pallas as pl
from jax.experimental.pallas import tpu as pltpu
from jax.experimental.pallas import tpu_sc as plsc

N = 10000
E = 320000
R = 8
NB = 4
D = 128

NC = 2
NS = 16
NW = NC * NS
CH = 128
NCHUNK = 79
EPW = CH * NCHUNK
E_PAD = EPW * NW
NCK0 = 104
NCK1 = 54
N_PAD = 10112
INVLEN = N * R + 16

_mesh = plsc.VectorSubcoreMesh(core_axis_name="c", subcore_axis_name="s")


def _worker_id():
    return lax.axis_index("s") * NC + lax.axis_index("c")


@functools.partial(
    pl.kernel,
    mesh=_mesh,
    compiler_params=pltpu.CompilerParams(needs_layout_passes=False),
    out_type=jax.ShapeDtypeStruct((NC, N_PAD, D), jnp.float32),
    scratch_types=[
        pltpu.VMEM((2, CH), jnp.int32),
        pltpu.VMEM((2, CH), jnp.int32),
        pltpu.VMEM((CH, D), jnp.float32),
        pltpu.VMEM((CH, D), jnp.float32),
        pltpu.VMEM_SHARED((N_PAD, D), jnp.float32),
        pltpu.SemaphoreType.DMA,
        pltpu.SemaphoreType.DMA,
    ],
)
def _sc_count(dst_hbm, et_hbm, zeros_hbm, out_hbm, dstb, etb, oneb0, oneb1, acc,
              ss0, ss1):
    core = lax.axis_index("c")
    sub = lax.axis_index("s")
    tile_row0 = sub * (N_PAD // NS)

    pltpu.sync_copy(zeros_hbm, oneb0)
    pltpu.sync_copy(zeros_hbm, oneb1)
    rows_per_sub = N_PAD // NS
    nfull = rows_per_sub // CH
    for j in range(nfull):
        pltpu.sync_copy(oneb0, acc.at[pl.ds(tile_row0 + j * CH, CH)])
    rem = rows_per_sub - nfull * CH
    if rem:
        pltpu.sync_copy(oneb0.at[pl.ds(0, rem)],
                        acc.at[pl.ds(tile_row0 + nfull * CH, rem)])
    plsc.subcore_barrier()

    mychunks = jnp.where(core == 0, NCK0, NCK1)
    base = jnp.where(core == 0, sub * NCK0, 16 * NCK0 + sub * NCK1)
    lane = lax.iota(jnp.int32, 16)

    def build(c, oneb, b):
        eoff = (base + c) * CH
        pltpu.sync_copy(dst_hbm.at[pl.ds(eoff, CH)], dstb.at[b])
        pltpu.sync_copy(et_hbm.at[pl.ds(eoff, CH)], etb.at[b])

        def grp(i, _):
            etv = etb[b, pl.ds(i * 16, 16)]
            for k in range(16):
                row = jnp.where(lane == etv[k], 1.0, 0.0)
                oneb[i * 16 + k, pl.ds(0, 16)] = row
            return _
        lax.fori_loop(0, CH // 16, grp, None)

    def scat(oneb, b, sem):
        pltpu.async_copy(oneb, acc.at[dstb.at[b]], sem, add=True)

    def swait(oneb, sem):
        pltpu.make_async_copy(oneb, acc.at[pl.ds(0, CH)], sem).wait()

    def pair_body(p, _):
        c0 = 2 * p
        c1 = 2 * p + 1

        @pl.when(c0 < mychunks)
        def _():
            build(c0, oneb0, 0)
            scat(oneb0, 0, ss0)

        @pl.when(c1 < mychunks)
        def _():
            build(c1, oneb1, 1)
            scat(oneb1, 1, ss1)

        @pl.when(c0 < mychunks)
        def _():
            swait(oneb0, ss0)

        @pl.when(c1 < mychunks)
        def _():
            swait(oneb1, ss1)
        return _

    lax.fori_loop(0, (max(NCK0, NCK1) + 1) // 2, pair_body, None)
    plsc.subcore_barrier()

    pltpu.sync_copy(acc.at[pl.ds(tile_row0, rows_per_sub)],
                    out_hbm.at[core, pl.ds(tile_row0, rows_per_sub)])


@functools.partial(
    pl.kernel,
    mesh=_mesh,
    compiler_params=pltpu.CompilerParams(needs_layout_passes=False),
    out_type=jax.ShapeDtypeStruct((E_PAD,), jnp.float32),
    scratch_types=[
        pltpu.VMEM((INVLEN,), jnp.float32),
        pltpu.VMEM((EPW,), jnp.int32),
        pltpu.VMEM((EPW,), jnp.int32),
        pltpu.VMEM((EPW,), jnp.float32),
    ],
)
def _sc_weights(dst_hbm, et_hbm, inv_hbm, out_hbm, invb, dstb, etb, wb):
    wid = _worker_id()
    eoff = wid * EPW
    pltpu.sync_copy(inv_hbm, invb)
    pltpu.sync_copy(dst_hbm.at[pl.ds(eoff, EPW)], dstb)
    pltpu.sync_copy(et_hbm.at[pl.ds(eoff, EPW)], etb)

    def body(i, _):
        dv = dstb[pl.ds(i * 16, 16)]
        tv = etb[pl.ds(i * 16, 16)]
        idx = dv * R + tv
        wb[pl.ds(i * 16, 16)] = plsc.load_gather(invb, [idx])
        return _

    lax.fori_loop(0, EPW // 16, body, None)
    pltpu.sync_copy(wb, out_hbm.at[pl.ds(eoff, EPW)])


@functools.partial(
    pl.kernel,
    mesh=_mesh,
    compiler_params=pltpu.CompilerParams(needs_layout_passes=False),
    out_type=jax.ShapeDtypeStruct((NC, N_PAD, D), jnp.float32),
    scratch_types=[
        pltpu.VMEM((2, CH), jnp.int32),
        pltpu.VMEM((2, CH), jnp.int32),
        pltpu.VMEM((2, CH), jnp.float32),
        pltpu.VMEM((CH, D), jnp.float32),
        pltpu.VMEM((CH, D), jnp.float32),
        pltpu.VMEM_SHARED((N_PAD, D), jnp.float32),
        pltpu.SemaphoreType.DMA,
        pltpu.SemaphoreType.DMA,
        pltpu.SemaphoreType.DMA,
        pltpu.SemaphoreType.DMA,
    ],
)
def _sc_agg(xw_hbm, gidx_hbm, dst_hbm, wgt_hbm, zeros_hbm, out_hbm,
            gb, db, wbf, rows0, rows1, acc, gs0, gs1, ss0, ss1):
    core = lax.axis_index("c")
    sub = lax.axis_index("s")
    wid = sub * NC + core
    tile_row0 = sub * (N_PAD // NS)

    pltpu.sync_copy(zeros_hbm, rows0)
    rows_per_sub = N_PAD // NS
    nfull = rows_per_sub // CH
    for j in range(nfull):
        pltpu.sync_copy(rows0, acc.at[pl.ds(tile_row0 + j * CH, CH)])
    rem = rows_per_sub - nfull * CH
    if rem:
        pltpu.sync_copy(rows0.at[pl.ds(0, rem)],
                        acc.at[pl.ds(tile_row0 + nfull * CH, rem)])
    plsc.subcore_barrier()

    mychunks = jnp.where(core == 0, NCK0, NCK1)
    base = jnp.where(core == 0, sub * NCK0, 16 * NCK0 + sub * NCK1)

    def meta(c, b):
        eoff = (base + c) * CH
        pltpu.sync_copy(gidx_hbm.at[pl.ds(eoff, CH)], gb.at[b])
        pltpu.sync_copy(dst_hbm.at[pl.ds(eoff, CH)], db.at[b])
        pltpu.sync_copy(wgt_hbm.at[pl.ds(eoff, CH)], wbf.at[b])

    def scale(b, rows):
        def grp(i, _):
            wv = wbf[b, pl.ds(i * 16, 16)]
            for k in range(16):
                w = wv[k]
                e = i * 16 + k
                for d in range(D // 16):
                    rows[e, pl.ds(d * 16, 16)] = rows[e, pl.ds(d * 16, 16)] * w
            return _
        lax.fori_loop(0, CH // 16, grp, None)

    def gather(b, rows, sem):
        pltpu.async_copy(xw_hbm.at[gb.at[b]], rows, sem)

    def scatter(b, rows, sem):
        pltpu.async_copy(rows, acc.at[db.at[b]], sem, add=True)

    def gwait(rows, sem):
        pltpu.make_async_copy(xw_hbm.at[pl.ds(0, CH)], rows, sem).wait()

    def swait(rows, sem):
        pltpu.make_async_copy(rows, acc.at[pl.ds(0, CH)], sem).wait()

    meta(0, 0)
    meta(1, 1)
    gather(0, rows0, gs0)
    gather(1, rows1, gs1)

    def pair_body(p, _):
        c0 = 2 * p
        c1 = 2 * p + 1

        @pl.when(c0 < mychunks)
        def _():
            gwait(rows0, gs0)
            scale(0, rows0)
            scatter(0, rows0, ss0)

        @pl.when(c1 < mychunks)
        def _():
            gwait(rows1, gs1)
            scale(1, rows1)
            scatter(1, rows1, ss1)

        @pl.when(c0 < mychunks)
        def _():
            swait(rows0, ss0)

            @pl.when(c0 + 2 < mychunks)
            def _():
                meta(c0 + 2, 0)
                gather(0, rows0, gs0)

        @pl.when(c1 < mychunks)
        def _():
            swait(rows1, ss1)

            @pl.when(c1 + 2 < mychunks)
            def _():
                meta(c1 + 2, 1)
                gather(1, rows1, gs1)
        return _

    lax.fori_loop(0, (max(NCK0, NCK1) + 1) // 2, pair_body, None)
    plsc.subcore_barrier()

    pltpu.sync_copy(acc.at[pl.ds(tile_row0, rows_per_sub)],
                    out_hbm.at[core, pl.ds(tile_row0, rows_per_sub)])


def _w_body(comp_ref, bases_ref, w_ref):
    comp = comp_ref[...]
    bases = bases_ref[...].reshape(NB, D * D)
    w_ref[...] = jnp.dot(comp, bases,
                         preferred_element_type=jnp.float32).reshape(R, D, D)


def _tc_w(comp, bases):
    return pl.pallas_call(
        _w_body,
        out_shape=jax.ShapeDtypeStruct((R, D, D), jnp.float32),
    )(comp, bases)


def _inv_body(cnt_ref, inv_ref):
    c = cnt_ref[0] + cnt_ref[1]
    inv_ref[...] = 1.0 / jnp.maximum(c[:, :R], 1.0)


def _tc_inv(cnt_parts):
    TN = 2000
    return pl.pallas_call(
        _inv_body,
        grid=(N // TN,),
        in_specs=[pl.BlockSpec((NC, TN, D), lambda i: (0, i, 0))],
        out_specs=pl.BlockSpec((TN, R), lambda i: (i, 0)),
        out_shape=jax.ShapeDtypeStruct((N, R), jnp.float32),
    )(cnt_parts)


_TN = 1000


def _a0_body(h_ref, root_ref, w_ref, bias_ref, ro_ref, xw_ref):
    h = h_ref[...]
    ro_ref[...] = jnp.dot(h, root_ref[...],
                          preferred_element_type=jnp.float32) + bias_ref[...]
    for r in range(R):
        xw_ref[r] = jnp.dot(h, w_ref[r], preferred_element_type=jnp.float32)


def _tc_a0(h, root, w, bias):
    return pl.pallas_call(
        _a0_body,
        grid=(N // _TN,),
        in_specs=[
            pl.BlockSpec((_TN, D), lambda i: (i, 0)),
            pl.BlockSpec((D, D), lambda i: (0, 0)),
            pl.BlockSpec((R, D, D), lambda i: (0, 0, 0)),
            pl.BlockSpec((D,), lambda i: (0,)),
        ],
        out_specs=[
            pl.BlockSpec((_TN, D), lambda i: (i, 0)),
            pl.BlockSpec((R, _TN, D), lambda i: (0, i, 0)),
        ],
        out_shape=[
            jax.ShapeDtypeStruct((N, D), jnp.float32),
            jax.ShapeDtypeStruct((R, N, D), jnp.float32),
        ],
    )(h, root, w, bias)


def _abn_body(pre_ref, stats_ref, gamma_ref, beta_ref, root_ref, w_ref,
              bias_ref, ro_ref, xw_ref):
    m = stats_ref[0] / N
    v = stats_ref[1] / N - m * m
    scale = gamma_ref[...] * lax.rsqrt(v + 1e-5)
    shift = beta_ref[...] - m * scale
    h = jnp.maximum(pre_ref[...] * scale + shift, 0.0)
    ro_ref[...] = jnp.dot(h, root_ref[...],
                          preferred_element_type=jnp.float32) + bias_ref[...]
    for r in range(R):
        xw_ref[r] = jnp.dot(h, w_ref[r], preferred_element_type=jnp.float32)


def _tc_abn(pre, stats, gamma, beta, root, w, bias):
    return pl.pallas_call(
        _abn_body,
        grid=(N // _TN,),
        in_specs=[
            pl.BlockSpec((_TN, D), lambda i: (i, 0)),
            pl.BlockSpec((2, D), lambda i: (0, 0)),
            pl.BlockSpec((D,), lambda i: (0,)),
            pl.BlockSpec((D,), lambda i: (0,)),
            pl.BlockSpec((D, D), lambda i: (0, 0)),
            pl.BlockSpec((R, D, D), lambda i: (0, 0, 0)),
            pl.BlockSpec((D,), lambda i: (0,)),
        ],
        out_specs=[
            pl.BlockSpec((_TN, D), lambda i: (i, 0)),
            pl.BlockSpec((R, _TN, D), lambda i: (0, i, 0)),
        ],
        out_shape=[
            jax.ShapeDtypeStruct((N, D), jnp.float32),
            jax.ShapeDtypeStruct((R, N, D), jnp.float32),
        ],
    )(pre, stats, gamma, beta, root, w, bias)


def _b01_body(ro_ref, parts_ref, pre_ref, stats_ref):
    i = pl.program_id(0)
    pre = ro_ref[...] + parts_ref[0] + parts_ref[1]
    pre_ref[...] = pre

    @pl.when(i == 0)
    def _():
        stats_ref[...] = jnp.zeros_like(stats_ref)

    stats_ref[0] += jnp.sum(pre, axis=0)
    stats_ref[1] += jnp.sum(pre * pre, axis=0)


def _tc_b01(ro, parts):
    return pl.pallas_call(
        _b01_body,
        grid=(N // _TN,),
        in_specs=[
            pl.BlockSpec((_TN, D), lambda i: (i, 0)),
            pl.BlockSpec((NC, _TN, D), lambda i: (0, i, 0)),
        ],
        out_specs=[
            pl.BlockSpec((_TN, D), lambda i: (i, 0)),
            pl.BlockSpec((2, D), lambda i: (0, 0)),
        ],
        out_shape=[
            jax.ShapeDtypeStruct((N, D), jnp.float32),
            jax.ShapeDtypeStruct((2, D), jnp.float32),
        ],
    )(ro, parts)


def _b2_body(ro_ref, parts_ref, out_ref):
    out_ref[...] = ro_ref[...] + parts_ref[0] + parts_ref[1]


def _tc_b2(ro, parts):
    return pl.pallas_call(
        _b2_body,
        grid=(N // _TN,),
        in_specs=[
            pl.BlockSpec((_TN, D), lambda i: (i, 0)),
            pl.BlockSpec((NC, _TN, D), lambda i: (0, i, 0)),
        ],
        out_specs=pl.BlockSpec((_TN, D), lambda i: (i, 0)),
        out_shape=jax.ShapeDtypeStruct((N, D), jnp.float32),
    )(ro, parts)


def kernel(x, edge_index, edge_type,
           comp0, bases0, root0, bias0,
           comp1, bases1, root1, bias1,
           comp2, bases2, root2, bias2,
           gamma0, beta0, gamma1, beta1):
    src = edge_index[0].astype(jnp.int32)
    dst = edge_index[1].astype(jnp.int32)
    et = edge_type.astype(jnp.int32)

    npad = E_PAD - E
    src_p = jnp.concatenate([src, jnp.zeros((npad,), jnp.int32)])
    dst_p = jnp.concatenate([dst, jnp.full((npad,), N, jnp.int32)])
    et_p = jnp.concatenate([et, jnp.zeros((npad,), jnp.int32)])
    gidx = et_p * N + src_p

    zrows = jnp.zeros((CH, D), jnp.float32)

    cnt_parts = _sc_count(dst_p, et_p, zrows)
    inv = _tc_inv(cnt_parts)
    inv_flat = jnp.concatenate([inv.reshape(N * R),
                                jnp.zeros((INVLEN - N * R,), jnp.float32)])
    wgt = _sc_weights(dst_p, et_p, inv_flat)

    w0 = _tc_w(comp0, bases0)
    w1 = _tc_w(comp1, bases1)
    w2 = _tc_w(comp2, bases2)

    ro0, xw0 = _tc_a0(x, root0, w0, bias0)
    parts0 = _sc_agg(xw0.reshape(R * N, D), gidx, dst_p, wgt, zrows)
    pre1, stats1 = _tc_b01(ro0, parts0)

    ro1, xw1 = _tc_abn(pre1, stats1, gamma0, beta0, root1, w1, bias1)
    parts1 = _sc_agg(xw1.reshape(R * N, D), gidx, dst_p, wgt, zrows)
    pre2, stats2 = _tc_b01(ro1, parts1)

    ro2, xw2 = _tc_abn(pre2, stats2, gamma1, beta1, root2, w2, bias2)
    parts2 = _sc_agg(xw2.reshape(R * N, D), gidx, dst_p, wgt, zrows)
    return _tc_b2(ro2, parts2)


# speedup vs baseline: 1.0829x; 1.0829x over previous
import functools

import jax
import jax.numpy as jnp
from jax import lax
from jax.experimental import pallas as pl
from jax.experimental.pallas import tpu as pltpu
from jax.experimental.pallas import tpu_sc as plsc

N = 10000
E = 320000
R = 8
NB = 4
D = 128

NC = 2
NS = 16
NW = NC * NS
CH = 128
NCHUNK = 79
EPW = CH * NCHUNK
E_PAD = EPW * NW
NCK0 = 104
NCK1 = 54
N_PAD = 10112
INVLEN = N * R + 16

_mesh = plsc.VectorSubcoreMesh(core_axis_name="c", subcore_axis_name="s")


def _worker_id():
    return lax.axis_index("s") * NC + lax.axis_index("c")


@functools.partial(
    pl.kernel,
    mesh=_mesh,
    compiler_params=pltpu.CompilerParams(needs_layout_passes=False),
    out_type=jax.ShapeDtypeStruct((NC, N_PAD, D), jnp.float32),
    scratch_types=[
        pltpu.VMEM((NCHUNK, CH), jnp.int32),
        pltpu.VMEM((2, CH), jnp.int32),
        pltpu.VMEM((CH, D), jnp.float32),
        pltpu.VMEM((CH, D), jnp.float32),
        pltpu.VMEM_SHARED((N_PAD, D), jnp.float32),
        pltpu.SemaphoreType.DMA,
        pltpu.SemaphoreType.DMA,
    ],
)
def _sc_count(dst_hbm, et_hbm, zeros_hbm, out_hbm, dstb, etb, oneb0, oneb1, acc,
              ss0, ss1):
    core = lax.axis_index("c")
    sub = lax.axis_index("s")
    wid = sub * NC + core
    tile_row0 = sub * (N_PAD // NS)

    pltpu.sync_copy(zeros_hbm, oneb0)
    pltpu.sync_copy(zeros_hbm, oneb1)
    rows_per_sub = N_PAD // NS
    nfull = rows_per_sub // CH
    for j in range(nfull):
        pltpu.sync_copy(oneb0, acc.at[pl.ds(tile_row0 + j * CH, CH)])
    rem = rows_per_sub - nfull * CH
    if rem:
        pltpu.sync_copy(oneb0.at[pl.ds(0, rem)],
                        acc.at[pl.ds(tile_row0 + nfull * CH, rem)])
    pltpu.sync_copy(dst_hbm.at[wid], dstb)
    plsc.subcore_barrier()

    lane = lax.iota(jnp.int32, 16)

    def build(c, oneb, b):
        pltpu.sync_copy(et_hbm.at[pl.ds(wid * EPW + c * CH, CH)], etb.at[b])

        def grp(i, _):
            etv = etb[b, pl.ds(i * 16, 16)]
            for k in range(16):
                row = jnp.where(lane == etv[k], 1.0, 0.0)
                oneb[i * 16 + k, pl.ds(0, 16)] = row
            return _
        lax.fori_loop(0, CH // 16, grp, None)

    def scat(c, oneb, sem):
        pltpu.async_copy(oneb, acc.at[dstb.at[c]], sem, add=True)

    def swait(oneb, sem):
        pltpu.make_async_copy(oneb, acc.at[pl.ds(0, CH)], sem).wait()

    build(0, oneb0, 0)
    scat(0, oneb0, ss0)

    def pair_body(p, _):
        c0 = 2 * p + 1
        c1 = 2 * p + 2
        build(c0, oneb1, 1)
        scat(c0, oneb1, ss1)
        swait(oneb0, ss0)

        @pl.when(c1 < NCHUNK)
        def _():
            build(c1, oneb0, 0)
            scat(c1, oneb0, ss0)
        swait(oneb1, ss1)
        return _

    lax.fori_loop(0, NCHUNK // 2, pair_body, None)
    swait(oneb0, ss0)
    plsc.subcore_barrier()

    pltpu.sync_copy(acc.at[pl.ds(tile_row0, rows_per_sub)],
                    out_hbm.at[core, pl.ds(tile_row0, rows_per_sub)])


@functools.partial(
    pl.kernel,
    mesh=_mesh,
    compiler_params=pltpu.CompilerParams(needs_layout_passes=False),
    out_type=jax.ShapeDtypeStruct((E_PAD,), jnp.float32),
    scratch_types=[
        pltpu.VMEM((INVLEN,), jnp.float32),
        pltpu.VMEM((EPW,), jnp.int32),
        pltpu.VMEM((EPW,), jnp.int32),
        pltpu.VMEM((EPW,), jnp.float32),
    ],
)
def _sc_weights(dst_hbm, et_hbm, inv_hbm, out_hbm, invb, dstb, etb, wb):
    wid = _worker_id()
    eoff = wid * EPW
    pltpu.sync_copy(inv_hbm, invb)
    pltpu.sync_copy(dst_hbm.at[pl.ds(eoff, EPW)], dstb)
    pltpu.sync_copy(et_hbm.at[pl.ds(eoff, EPW)], etb)

    def body(i, _):
        dv = dstb[pl.ds(i * 16, 16)]
        tv = etb[pl.ds(i * 16, 16)]
        idx = dv * R + tv
        wb[pl.ds(i * 16, 16)] = plsc.load_gather(invb, [idx])
        return _

    lax.fori_loop(0, EPW // 16, body, None)
    pltpu.sync_copy(wb, out_hbm.at[pl.ds(eoff, EPW)])


@functools.partial(
    pl.kernel,
    mesh=_mesh,
    compiler_params=pltpu.CompilerParams(needs_layout_passes=False),
    out_type=jax.ShapeDtypeStruct((NC, N_PAD, D), jnp.float32),
    scratch_types=[
        pltpu.VMEM((2, CH), jnp.int32),
        pltpu.VMEM((2, CH), jnp.int32),
        pltpu.VMEM((2, CH), jnp.float32),
        pltpu.VMEM((CH, D), jnp.float32),
        pltpu.VMEM((CH, D), jnp.float32),
        pltpu.VMEM_SHARED((N_PAD, D), jnp.float32),
        pltpu.SemaphoreType.DMA,
        pltpu.SemaphoreType.DMA,
        pltpu.SemaphoreType.DMA,
        pltpu.SemaphoreType.DMA,
    ],
)
def _sc_agg(xw_hbm, gidx_hbm, dst_hbm, wgt_hbm, zeros_hbm, out_hbm,
            gb, db, wbf, rows0, rows1, acc, gs0, gs1, ss0, ss1):
    core = lax.axis_index("c")
    sub = lax.axis_index("s")
    wid = sub * NC + core
    tile_row0 = sub * (N_PAD // NS)

    pltpu.sync_copy(zeros_hbm, rows0)
    rows_per_sub = N_PAD // NS
    nfull = rows_per_sub // CH
    for j in range(nfull):
        pltpu.sync_copy(rows0, acc.at[pl.ds(tile_row0 + j * CH, CH)])
    rem = rows_per_sub - nfull * CH
    if rem:
        pltpu.sync_copy(rows0.at[pl.ds(0, rem)],
                        acc.at[pl.ds(tile_row0 + nfull * CH, rem)])
    plsc.subcore_barrier()

    mychunks = jnp.where(core == 0, NCK0, NCK1)
    base = jnp.where(core == 0, sub * NCK0, 16 * NCK0 + sub * NCK1)

    def meta(c, b):
        eoff = (base + c) * CH
        pltpu.sync_copy(gidx_hbm.at[pl.ds(eoff, CH)], gb.at[b])
        pltpu.sync_copy(dst_hbm.at[pl.ds(eoff, CH)], db.at[b])
        pltpu.sync_copy(wgt_hbm.at[pl.ds(eoff, CH)], wbf.at[b])

    def scale(b, rows):
        def grp(i, _):
            wv = wbf[b, pl.ds(i * 16, 16)]
            for k in range(16):
                w = wv[k]
                e = i * 16 + k
                for d in range(D // 16):
                    rows[e, pl.ds(d * 16, 16)] = rows[e, pl.ds(d * 16, 16)] * w
            return _
        lax.fori_loop(0, CH // 16, grp, None)

    def gather(b, rows, sem):
        pltpu.async_copy(xw_hbm.at[gb.at[b]], rows, sem)

    def scatter(b, rows, sem):
        pltpu.async_copy(rows, acc.at[db.at[b]], sem, add=True)

    def gwait(rows, sem):
        pltpu.make_async_copy(xw_hbm.at[pl.ds(0, CH)], rows, sem).wait()

    def swait(rows, sem):
        pltpu.make_async_copy(rows, acc.at[pl.ds(0, CH)], sem).wait()

    meta(0, 0)
    meta(1, 1)
    gather(0, rows0, gs0)
    gather(1, rows1, gs1)

    def pair_body(p, _):
        c0 = 2 * p
        c1 = 2 * p + 1

        @pl.when(c0 < mychunks)
        def _():
            gwait(rows0, gs0)
            scale(0, rows0)
            scatter(0, rows0, ss0)

        @pl.when(c1 < mychunks)
        def _():
            gwait(rows1, gs1)
            scale(1, rows1)
            scatter(1, rows1, ss1)

        @pl.when(c0 < mychunks)
        def _():
            swait(rows0, ss0)

            @pl.when(c0 + 2 < mychunks)
            def _():
                meta(c0 + 2, 0)
                gather(0, rows0, gs0)

        @pl.when(c1 < mychunks)
        def _():
            swait(rows1, ss1)

            @pl.when(c1 + 2 < mychunks)
            def _():
                meta(c1 + 2, 1)
                gather(1, rows1, gs1)
        return _

    lax.fori_loop(0, (max(NCK0, NCK1) + 1) // 2, pair_body, None)
    plsc.subcore_barrier()

    pltpu.sync_copy(acc.at[pl.ds(tile_row0, rows_per_sub)],
                    out_hbm.at[core, pl.ds(tile_row0, rows_per_sub)])


def _w_body(comp_ref, bases_ref, w_ref):
    comp = comp_ref[...]
    bases = bases_ref[...].reshape(NB, D * D)
    w_ref[...] = jnp.dot(comp, bases,
                         preferred_element_type=jnp.float32).reshape(R, D, D)


def _tc_w(comp, bases):
    return pl.pallas_call(
        _w_body,
        out_shape=jax.ShapeDtypeStruct((R, D, D), jnp.float32),
    )(comp, bases)


def _inv_body(cnt_ref, inv_ref):
    c = cnt_ref[0] + cnt_ref[1]
    inv_ref[...] = 1.0 / jnp.maximum(c[:, :R], 1.0)


def _tc_inv(cnt_parts):
    TN = 2000
    return pl.pallas_call(
        _inv_body,
        grid=(N // TN,),
        in_specs=[pl.BlockSpec((NC, TN, D), lambda i: (0, i, 0))],
        out_specs=pl.BlockSpec((TN, R), lambda i: (i, 0)),
        out_shape=jax.ShapeDtypeStruct((N, R), jnp.float32),
    )(cnt_parts)


_TN = 1000


def _a0_body(h_ref, root_ref, w_ref, bias_ref, ro_ref, xw_ref):
    h = h_ref[...]
    ro_ref[...] = jnp.dot(h, root_ref[...],
                          preferred_element_type=jnp.float32) + bias_ref[...]
    for r in range(R):
        xw_ref[r] = jnp.dot(h, w_ref[r], preferred_element_type=jnp.float32)


def _tc_a0(h, root, w, bias):
    return pl.pallas_call(
        _a0_body,
        grid=(N // _TN,),
        in_specs=[
            pl.BlockSpec((_TN, D), lambda i: (i, 0)),
            pl.BlockSpec((D, D), lambda i: (0, 0)),
            pl.BlockSpec((R, D, D), lambda i: (0, 0, 0)),
            pl.BlockSpec((D,), lambda i: (0,)),
        ],
        out_specs=[
            pl.BlockSpec((_TN, D), lambda i: (i, 0)),
            pl.BlockSpec((R, _TN, D), lambda i: (0, i, 0)),
        ],
        out_shape=[
            jax.ShapeDtypeStruct((N, D), jnp.float32),
            jax.ShapeDtypeStruct((R, N, D), jnp.float32),
        ],
    )(h, root, w, bias)


def _abn_body(pre_ref, stats_ref, gamma_ref, beta_ref, root_ref, w_ref,
              bias_ref, ro_ref, xw_ref):
    m = stats_ref[0] / N
    v = stats_ref[1] / N - m * m
    scale = gamma_ref[...] * lax.rsqrt(v + 1e-5)
    shift = beta_ref[...] - m * scale
    h = jnp.maximum(pre_ref[...] * scale + shift, 0.0)
    ro_ref[...] = jnp.dot(h, root_ref[...],
                          preferred_element_type=jnp.float32) + bias_ref[...]
    for r in range(R):
        xw_ref[r] = jnp.dot(h, w_ref[r], preferred_element_type=jnp.float32)


def _tc_abn(pre, stats, gamma, beta, root, w, bias):
    return pl.pallas_call(
        _abn_body,
        grid=(N // _TN,),
        in_specs=[
            pl.BlockSpec((_TN, D), lambda i: (i, 0)),
            pl.BlockSpec((2, D), lambda i: (0, 0)),
            pl.BlockSpec((D,), lambda i: (0,)),
            pl.BlockSpec((D,), lambda i: (0,)),
            pl.BlockSpec((D, D), lambda i: (0, 0)),
            pl.BlockSpec((R, D, D), lambda i: (0, 0, 0)),
            pl.BlockSpec((D,), lambda i: (0,)),
        ],
        out_specs=[
            pl.BlockSpec((_TN, D), lambda i: (i, 0)),
            pl.BlockSpec((R, _TN, D), lambda i: (0, i, 0)),
        ],
        out_shape=[
            jax.ShapeDtypeStruct((N, D), jnp.float32),
            jax.ShapeDtypeStruct((R, N, D), jnp.float32),
        ],
    )(pre, stats, gamma, beta, root, w, bias)


def _b01_body(ro_ref, parts_ref, pre_ref, stats_ref):
    i = pl.program_id(0)
    pre = ro_ref[...] + parts_ref[0] + parts_ref[1]
    pre_ref[...] = pre

    @pl.when(i == 0)
    def _():
        stats_ref[...] = jnp.zeros_like(stats_ref)

    stats_ref[0] += jnp.sum(pre, axis=0)
    stats_ref[1] += jnp.sum(pre * pre, axis=0)


def _tc_b01(ro, parts):
    return pl.pallas_call(
        _b01_body,
        grid=(N // _TN,),
        in_specs=[
            pl.BlockSpec((_TN, D), lambda i: (i, 0)),
            pl.BlockSpec((NC, _TN, D), lambda i: (0, i, 0)),
        ],
        out_specs=[
            pl.BlockSpec((_TN, D), lambda i: (i, 0)),
            pl.BlockSpec((2, D), lambda i: (0, 0)),
        ],
        out_shape=[
            jax.ShapeDtypeStruct((N, D), jnp.float32),
            jax.ShapeDtypeStruct((2, D), jnp.float32),
        ],
    )(ro, parts)


def _b2_body(ro_ref, parts_ref, out_ref):
    out_ref[...] = ro_ref[...] + parts_ref[0] + parts_ref[1]


def _tc_b2(ro, parts):
    return pl.pallas_call(
        _b2_body,
        grid=(N // _TN,),
        in_specs=[
            pl.BlockSpec((_TN, D), lambda i: (i, 0)),
            pl.BlockSpec((NC, _TN, D), lambda i: (0, i, 0)),
        ],
        out_specs=pl.BlockSpec((_TN, D), lambda i: (i, 0)),
        out_shape=jax.ShapeDtypeStruct((N, D), jnp.float32),
    )(ro, parts)


def kernel(x, edge_index, edge_type,
           comp0, bases0, root0, bias0,
           comp1, bases1, root1, bias1,
           comp2, bases2, root2, bias2,
           gamma0, beta0, gamma1, beta1):
    src = edge_index[0].astype(jnp.int32)
    dst = edge_index[1].astype(jnp.int32)
    et = edge_type.astype(jnp.int32)

    npad = E_PAD - E
    src_p = jnp.concatenate([src, jnp.zeros((npad,), jnp.int32)])
    dst_p = jnp.concatenate([dst, jnp.full((npad,), N, jnp.int32)])
    et_p = jnp.concatenate([et, jnp.zeros((npad,), jnp.int32)])
    gidx = et_p * N + src_p

    zrows = jnp.zeros((CH, D), jnp.float32)

    dst3 = dst_p.reshape(NW, NCHUNK, CH)
    cnt_parts = _sc_count(dst3, et_p, zrows)
    inv = _tc_inv(cnt_parts)
    inv_flat = jnp.concatenate([inv.reshape(N * R),
                                jnp.zeros((INVLEN - N * R,), jnp.float32)])
    wgt = _sc_weights(dst_p, et_p, inv_flat)

    w0 = _tc_w(comp0, bases0)
    w1 = _tc_w(comp1, bases1)
    w2 = _tc_w(comp2, bases2)

    ro0, xw0 = _tc_a0(x, root0, w0, bias0)
    parts0 = _sc_agg(xw0.reshape(R * N, D), gidx, dst_p, wgt, zrows)
    pre1, stats1 = _tc_b01(ro0, parts0)

    ro1, xw1 = _tc_abn(pre1, stats1, gamma0, beta0, root1, w1, bias1)
    parts1 = _sc_agg(xw1.reshape(R * N, D), gidx, dst_p, wgt, zrows)
    pre2, stats2 = _tc_b01(ro1, parts1)

    ro2, xw2 = _tc_abn(pre2, stats2, gamma1, beta1, root2, w2, bias2)
    parts2 = _sc_agg(xw2.reshape(R * N, D), gidx, dst_p, wgt, zrows)
    return _tc_b2(ro2, parts2)
